# baseline (speedup 1.0000x reference)
import functools

import jax
import jax.numpy as jnp
from jax import lax
from jax.experimental import pallas as pl
from jax.experimental.pallas import tpu as pltpu
from jax.experimental.pallas import tpu_sc as plsc

VOCAB = 100000
EMBED = 64
B = 4096
T = 200
BFLAT = B * T


@functools.cache
def _build(num_cores: int, num_subcores: int):
    nw = num_cores * num_subcores
    n_per_w = BFLAT // nw
    g = 128
    n_groups = n_per_w // g

    mesh = plsc.VectorSubcoreMesh(core_axis_name="c", subcore_axis_name="s")

    @functools.partial(
        pl.kernel,
        out_type=jax.ShapeDtypeStruct((BFLAT, EMBED), jnp.float32),
        mesh=mesh,
        scratch_types=[
            pltpu.VMEM((n_per_w,), jnp.int32),
            *([pltpu.VMEM((g, EMBED), jnp.float32)] * 4),
            *([pltpu.SemaphoreType.DMA] * 4),
        ],
        compiler_params=pltpu.CompilerParams(use_tc_tiling_on_sc=False),
    )
    def gather_kernel(tok_hbm, table_hbm, out_hbm, idx_v, b0, b1, b2, b3,
                      s0, s1, s2, s3):
        bufs = (b0, b1, b2, b3)
        sems = (s0, s1, s2, s3)
        wid = lax.axis_index("s") * num_cores + lax.axis_index("c")
        base = wid * n_per_w
        pltpu.sync_copy(tok_hbm.at[pl.ds(base, n_per_w)], idx_v)

        def fire(gi, buf, sem):
            pltpu.async_copy(
                table_hbm.at[idx_v.at[pl.ds(gi * g, g)]], buf, sem)

        def drain(buf, sem):
            pltpu.make_async_copy(
                table_hbm.at[idx_v.at[pl.ds(0, g)]], buf, sem).wait()

        def store(gi, buf):
            pltpu.sync_copy(buf, out_hbm.at[pl.ds(base + gi * g, g)])

        for j in range(4):
            fire(j, bufs[j], sems[j])

        @pl.loop(0, n_groups, step=4)
        def _(gi):
            for j in range(4):
                drain(bufs[j], sems[j])
                store(gi + j, bufs[j])

                @pl.when(gi + j + 4 < n_groups)
                def _():
                    fire(gi + j + 4, bufs[j], sems[j])

    return gather_kernel


def kernel(token_ids, table):
    info = plsc.get_sparse_core_info()
    fn = _build(info.num_cores, info.num_subcores)
    tok = token_ids.astype(jnp.int32).reshape(-1)
    out = fn(tok, table)
    out = lax.optimization_barrier(out)
    return out.reshape(B, T, EMBED)


# speedup vs baseline: 1.3031x; 1.3031x over previous
import functools

import jax
import jax.numpy as jnp
from jax import lax
from jax.experimental import pallas as pl
from jax.experimental.pallas import tpu as pltpu
from jax.experimental.pallas import tpu_sc as plsc

VOCAB = 100000
EMBED = 64
B = 4096
T = 200
BFLAT = B * T
ROW = 2 * EMBED
NBUF = 2


@functools.cache
def _build(num_cores: int, num_subcores: int):
    nw = num_cores * num_subcores
    n_per_w = BFLAT // nw
    g = 256
    n_groups = n_per_w // g

    mesh = plsc.VectorSubcoreMesh(core_axis_name="c", subcore_axis_name="s")

    @functools.partial(
        pl.kernel,
        out_type=jax.ShapeDtypeStruct((BFLAT, ROW), jnp.float32),
        mesh=mesh,
        scratch_types=[
            pltpu.VMEM((n_per_w,), jnp.int32),
            *([pltpu.VMEM((g, ROW), jnp.float32)] * NBUF),
            *([pltpu.SemaphoreType.DMA] * NBUF),
        ],
    )
    def gather_kernel(tok_hbm, table_hbm, out_hbm, idx_v, *rest):
        bufs = rest[:NBUF]
        sems = rest[NBUF:]
        wid = lax.axis_index("s") * num_cores + lax.axis_index("c")
        base = wid * n_per_w
        pltpu.sync_copy(tok_hbm.at[pl.ds(base, n_per_w)], idx_v)

        def fire(gi, buf, sem):
            for h in range(g // 128):
                pltpu.async_copy(
                    table_hbm.at[idx_v.at[pl.ds(gi * g + h * 128, 128)]],
                    buf.at[pl.ds(h * 128, 128)], sem)

        def drain(buf, sem):
            for h in range(g // 128):
                pltpu.make_async_copy(
                    table_hbm.at[idx_v.at[pl.ds(0, 128)]],
                    buf.at[pl.ds(h * 128, 128)], sem).wait()

        def store(gi, buf):
            pltpu.sync_copy(buf, out_hbm.at[pl.ds(base + gi * g, g)])

        for j in range(NBUF):
            fire(j, bufs[j], sems[j])

        @pl.loop(0, n_groups, step=NBUF)
        def _(gi):
            for j in range(NBUF):
                drain(bufs[j], sems[j])
                store(gi + j, bufs[j])

                @pl.when(gi + j + NBUF < n_groups)
                def _():
                    fire(gi + j + NBUF, bufs[j], sems[j])

    return gather_kernel


def kernel(token_ids, table):
    info = plsc.get_sparse_core_info()
    fn = _build(info.num_cores, info.num_subcores)
    tok = token_ids.astype(jnp.int32).reshape(-1)
    table_padded = jnp.pad(table, ((0, 0), (0, ROW - EMBED)))
    out = fn(tok, table_padded)
    return out[:, :EMBED].reshape(B, T, EMBED)


# speedup vs baseline: 1.3044x; 1.0010x over previous
import functools

import jax
import jax.numpy as jnp
from jax import lax
from jax.experimental import pallas as pl
from jax.experimental.pallas import tpu as pltpu
from jax.experimental.pallas import tpu_sc as plsc

VOCAB = 100000
EMBED = 64
B = 4096
T = 200
BFLAT = B * T
ROW = 2 * EMBED
NBUF = 2


@functools.cache
def _build(num_cores: int, num_subcores: int):
    nw = num_cores * num_subcores
    n_per_w = BFLAT // nw
    g = 256
    n_groups = n_per_w // g

    mesh = plsc.VectorSubcoreMesh(core_axis_name="c", subcore_axis_name="s")

    @functools.partial(
        pl.kernel,
        out_type=jax.ShapeDtypeStruct((BFLAT, ROW), jnp.float32),
        mesh=mesh,
        scratch_types=[
            pltpu.VMEM((n_per_w // 128, 128), jnp.int32),
            *([pltpu.VMEM((g, ROW), jnp.float32)] * NBUF),
            *([pltpu.SemaphoreType.DMA] * NBUF),
        ],
    )
    def gather_kernel(tok_hbm, table_hbm, out_hbm, idx_v, *rest):
        bufs = rest[:NBUF]
        sems = rest[NBUF:]
        wid = lax.axis_index("s") * num_cores + lax.axis_index("c")
        base = wid * n_per_w
        chunks_w = n_per_w // 128
        pltpu.sync_copy(tok_hbm.at[pl.ds(wid * chunks_w, chunks_w)], idx_v)

        def fire(gi, buf, sem):
            for h in range(g // 128):
                pltpu.async_copy(
                    table_hbm.at[idx_v.at[gi * (g // 128) + h]],
                    buf.at[pl.ds(h * 128, 128)], sem)

        def drain(buf, sem):
            for h in range(g // 128):
                pltpu.make_async_copy(
                    table_hbm.at[idx_v.at[0]],
                    buf.at[pl.ds(h * 128, 128)], sem).wait()

        def store(gi, buf):
            pltpu.sync_copy(buf, out_hbm.at[pl.ds(base + gi * g, g)])

        for j in range(NBUF):
            fire(j, bufs[j], sems[j])

        @pl.loop(0, n_groups, step=NBUF)
        def _(gi):
            for j in range(NBUF):
                drain(bufs[j], sems[j])
                store(gi + j, bufs[j])

                @pl.when(gi + j + NBUF < n_groups)
                def _():
                    fire(gi + j + NBUF, bufs[j], sems[j])

    return gather_kernel


def kernel(token_ids, table):
    info = plsc.get_sparse_core_info()
    fn = _build(info.num_cores, info.num_subcores)
    tok = token_ids.astype(jnp.int32).reshape(-1, 128)
    table_padded = jnp.pad(table, ((0, 0), (0, ROW - EMBED)))
    out = fn(tok, table_padded)
    return out[:, :EMBED].reshape(B, T, EMBED)
